# Initial kernel scaffold; baseline (speedup 1.0000x reference)
import jax
import jax.numpy as jnp
from jax.experimental import pallas as pl


def kernel(feature, xyz, bw1, bb1, bw2, bb2, bws, bbs, wq_w, wq_b, wk_w, wk_b, wv_w, wv_b, dw1, db1, dg1, dbe1, dw2, db2, gw1, gb1, gg1, gbe1, gw2, gb2, aw, ab, m4w1, m4b1, m4w2, m4b2):
    raise NotImplementedError("write your pallas kernel here")


# speedup vs baseline: 295.7408x; 295.7408x over previous
import functools

import jax
import jax.numpy as jnp
from jax import lax
from jax.experimental import pallas as pl
from jax.experimental.pallas import tpu as pltpu
from jax.experimental.pallas import tpu_sc as plsc

B, N, K, C_IN, D = 4, 2048, 16, 480, 64
KP1 = K + 1
TQ = 256
TM = 512
TA = 256
NW = 32
CHUNK = 128
TBL = 256


def _knn_body(xyz_q_ref, xyz_r_ref, out_ref):
    b = pl.program_id(0)
    q = xyz_q_ref[0]
    r = xyz_r_ref[0]
    sq_q = jnp.sum(q * q, axis=1, keepdims=True)
    sq_r = jnp.sum(r * r, axis=0, keepdims=True)
    inner = jax.lax.dot_general(
        q, r, dimension_numbers=(((1,), (0,)), ((), ())),
        preferred_element_type=jnp.float32,
        precision=jax.lax.Precision.HIGHEST)
    d2 = sq_q + sq_r - 2.0 * inner
    col = jax.lax.broadcasted_iota(jnp.int32, (TQ, N), 1)
    big = jnp.int32(N)
    inf = jnp.float32(jnp.inf)
    picks = []
    for rank in range(KP1):
        m = jnp.min(d2, axis=1, keepdims=True)
        eq = d2 <= m
        am = jnp.min(jnp.where(eq, col, big), axis=1)
        if rank > 0:
            picks.append(am[:, None])
        d2 = jnp.where(col == am[:, None], inf, d2)
    idx = jnp.concatenate(picks, axis=1)
    out_ref[0] = idx + b * N


def _knn(xyz_pm, xyz_cm):
    return pl.pallas_call(
        _knn_body,
        grid=(B, N // TQ),
        in_specs=[
            pl.BlockSpec((1, TQ, 8), lambda b, t: (b, t, 0)),
            pl.BlockSpec((1, 8, N), lambda b, t: (b, 0, 0)),
        ],
        out_specs=pl.BlockSpec((1, TQ, K), lambda b, t: (b, t, 0)),
        out_shape=jax.ShapeDtypeStruct((B, N, K), jnp.int32),
        compiler_params=pltpu.CompilerParams(
            dimension_semantics=("parallel", "parallel")),
    )(xyz_pm, xyz_cm)


def _mlp_body(ft_ref, xyzp_ref, w1_ref, b1_ref, w2_ref, b2_ref, ws_ref,
              bs_ref, wq_ref, bq_ref, wk_ref, bk_ref, wv_ref, bv_ref,
              dw1t_ref, x_ref, q_ref, tbl_ref):
    ft = ft_ref[0]
    h = jnp.maximum(
        jnp.dot(ft, w1_ref[...], preferred_element_type=jnp.float32)
        + b1_ref[...], 0.0)
    x = (jnp.dot(h, w2_ref[...], preferred_element_type=jnp.float32)
         + b2_ref[...]
         + jnp.dot(ft, ws_ref[...], preferred_element_type=jnp.float32)
         + bs_ref[...])
    q = jnp.dot(x, wq_ref[...], preferred_element_type=jnp.float32) + bq_ref[...]
    k = jnp.dot(x, wk_ref[...], preferred_element_type=jnp.float32) + bk_ref[...]
    v = jnp.dot(x, wv_ref[...], preferred_element_type=jnp.float32) + bv_ref[...]
    p = jnp.dot(xyzp_ref[0], dw1t_ref[...], preferred_element_type=jnp.float32)
    x_ref[0] = x
    q_ref[0] = q
    tbl_ref[0] = jnp.concatenate(
        [k, v, p, jnp.zeros((TM, D), jnp.float32)], axis=1)


def _mlp_qkv(feature_pm, xyz_p16, weights):
    full = lambda shape: pl.BlockSpec(shape, lambda b, t: tuple(0 for _ in shape))
    wspecs = []
    for w in weights:
        wspecs.append(full(w.shape))
    return pl.pallas_call(
        _mlp_body,
        grid=(B, N // TM),
        in_specs=[
            pl.BlockSpec((1, TM, C_IN), lambda b, t: (b, t, 0)),
            pl.BlockSpec((1, TM, 16), lambda b, t: (b, t, 0)),
        ] + wspecs,
        out_specs=[
            pl.BlockSpec((1, TM, D), lambda b, t: (b, t, 0)),
            pl.BlockSpec((1, TM, D), lambda b, t: (b, t, 0)),
            pl.BlockSpec((1, TM, TBL), lambda b, t: (b, t, 0)),
        ],
        out_shape=[
            jax.ShapeDtypeStruct((B, N, D), jnp.float32),
            jax.ShapeDtypeStruct((B, N, D), jnp.float32),
            jax.ShapeDtypeStruct((B, N, TBL), jnp.float32),
        ],
        compiler_params=pltpu.CompilerParams(
            dimension_semantics=("parallel", "parallel")),
    )(feature_pm, xyz_p16, *weights)


def _sc_gather(table, flat_idx):
    rows = B * N * K
    per_w = rows // NW
    n_chunks = per_w // CHUNK
    mesh = plsc.VectorSubcoreMesh(core_axis_name="c", subcore_axis_name="s")

    @functools.partial(
        pl.kernel,
        mesh=mesh,
        out_type=jax.ShapeDtypeStruct((rows, TBL), jnp.float32),
        scratch_types=[
            pltpu.VMEM((CHUNK,), jnp.int32),
            pltpu.VMEM((CHUNK, TBL), jnp.float32),
            pltpu.SemaphoreType.DMA,
        ],
    )
    def gather_k(table_hbm, idx_hbm, out_hbm, idx_v, rows_v, sem):
        wid = lax.axis_index("s") * 2 + lax.axis_index("c")
        base = wid * per_w

        def body(i, _):
            off = base + i * CHUNK
            pltpu.sync_copy(idx_hbm.at[pl.ds(off, CHUNK)], idx_v)
            pltpu.async_copy(table_hbm.at[idx_v], rows_v, sem).wait()
            pltpu.sync_copy(rows_v, out_hbm.at[pl.ds(off, CHUNK)])
            return ()

        lax.fori_loop(0, n_chunks, body, (), unroll=False)

    return gather_k(table, flat_idx)


def _attn_body(g_ref, q_ref, x_ref, p_ref, db1_ref, dw2_ref,
               db2_ref, gw1_ref, gb1_ref, gw2_ref, gb2_ref, awr_ref, ab_ref,
               m1_ref, mb1_ref, m2_ref, mb2_ref, res_ref, comp_ref):
    M = TA * K
    g = g_ref[0]
    k_l = g[:, :D]
    v_l = g[:, D:2 * D]
    p_l = g[:, 2 * D:3 * D]
    q = q_ref[0]
    x = x_ref[0]
    pq = p_ref[0]

    qb = jnp.broadcast_to(q[:, None, :], (TA, K, D)).reshape(M, D)
    pqb = jnp.broadcast_to(pq[:, None, :], (TA, K, D)).reshape(M, D)

    h = jnp.maximum(pqb - p_l + db1_ref[...], 0.0)
    pos = (jnp.dot(h, dw2_ref[...], preferred_element_type=jnp.float32)
           + db2_ref[...])

    s = qb - k_l + pos
    h2 = jnp.maximum(
        jnp.dot(s, gw1_ref[...], preferred_element_type=jnp.float32)
        + gb1_ref[...], 0.0)
    attn = (jnp.dot(h2, gw2_ref[...], preferred_element_type=jnp.float32)
            + gb2_ref[...])
    value = v_l + pos

    outs = []
    for r in range(4):
        t = (jnp.dot(attn, awr_ref[r], preferred_element_type=jnp.float32)
             + ab_ref[...])
        t = t - jnp.max(t, axis=1, keepdims=True)
        e = jnp.exp(t)
        p = e / jnp.sum(e, axis=1, keepdims=True)
        z = (p * value).reshape(TA, K, D)
        outs.append(jnp.sum(z, axis=1) + x)
    res = jnp.stack(outs, axis=1).reshape(TA * 4, D)
    res_ref[0] = res

    c = jnp.maximum(
        jnp.dot(res, m1_ref[...], preferred_element_type=jnp.float32)
        + mb1_ref[...], 0.0)
    comp_ref[0] = (jnp.dot(c, m2_ref[...], preferred_element_type=jnp.float32)
                   + mb2_ref[...])


def _attn(gathered, q_pm, x_pm, p_pm, weights):
    full = lambda shape: pl.BlockSpec(shape, lambda b, t: tuple(0 for _ in shape))
    wspecs = [full(w.shape) for w in weights]
    return pl.pallas_call(
        _attn_body,
        grid=(B, N // TA),
        in_specs=[
            pl.BlockSpec((1, TA * K, TBL), lambda b, t: (b, t, 0)),
            pl.BlockSpec((1, TA, D), lambda b, t: (b, t, 0)),
            pl.BlockSpec((1, TA, D), lambda b, t: (b, t, 0)),
            pl.BlockSpec((1, TA, D), lambda b, t: (b, t, 0)),
        ] + wspecs,
        out_specs=[
            pl.BlockSpec((1, TA * 4, D), lambda b, t: (b, t, 0)),
            pl.BlockSpec((1, TA * 4, 8), lambda b, t: (b, t, 0)),
        ],
        out_shape=[
            jax.ShapeDtypeStruct((B, 4 * N, D), jnp.float32),
            jax.ShapeDtypeStruct((B, 4 * N, 8), jnp.float32),
        ],
        compiler_params=pltpu.CompilerParams(
            dimension_semantics=("parallel", "parallel")),
    )(gathered, q_pm, x_pm, p_pm, *weights)


def kernel(feature, xyz, bw1, bb1, bw2, bb2, bws, bbs, wq_w, wq_b, wk_w, wk_b,
           wv_w, wv_b, dw1, db1, dg1, dbe1, dw2, db2, gw1, gb1, gg1, gbe1,
           gw2, gb2, aw, ab, m4w1, m4b1, m4w2, m4b2):
    f32 = jnp.float32
    eps = 1e-5

    feature_pm = jnp.transpose(feature, (0, 2, 1))
    xyz_pm = jnp.transpose(xyz, (0, 2, 1))
    xyz_p16 = jnp.concatenate(
        [xyz_pm, jnp.zeros((B, N, 13), f32)], axis=2)
    xyz_p8 = xyz_p16[:, :, :8]
    xyz_cm = jnp.concatenate(
        [xyz, jnp.zeros((B, 5, N), f32)], axis=1)

    s1 = dg1 / jnp.sqrt(1.0 + eps)
    sg = gg1 / jnp.sqrt(1.0 + eps)
    dw1t = jnp.concatenate(
        [dw1.T, jnp.zeros((13, D), f32)], axis=0) * s1[None, :]
    db1f = db1 * s1 + dbe1
    gw1t = gw1.T * sg[None, :]
    gb1f = gb1 * sg + gbe1
    awr = jnp.transpose(aw, (2, 0, 1))
    m4w2t = jnp.concatenate(
        [m4w2.T, jnp.zeros((D, 5), f32)], axis=1)
    m4b2p = jnp.concatenate([m4b2, jnp.zeros((5,), f32)])

    idx = _knn(xyz_p8, xyz_cm)
    x_pm, q_pm, table = _mlp_qkv(
        feature_pm, xyz_p16,
        [bw1.T, bb1[None, :], bw2.T, bb2[None, :], bws.T, bbs[None, :],
         wq_w.T, wq_b[None, :], wk_w.T, wk_b[None, :], wv_w.T, wv_b[None, :],
         dw1t])
    p_pm = table[:, :, 2 * D:3 * D]

    gathered = _sc_gather(table.reshape(B * N, TBL), idx.reshape(B * N * K))
    gathered = gathered.reshape(B, N * K, TBL)

    res_pm, comp_pm = _attn(
        gathered, q_pm, x_pm, p_pm,
        [db1f[None, :], dw2.T, db2[None, :], gw1t, gb1f[None, :],
         gw2.T, gb2[None, :], awr, ab[None, :], m4w1.T, m4b1[None, :],
         m4w2t, m4b2p[None, :]])

    res = jnp.transpose(res_pm, (0, 2, 1))
    completion = jnp.transpose(comp_pm[:, :, :3], (0, 2, 1))
    return (completion, res)


# speedup vs baseline: 332.8903x; 1.1256x over previous
import functools

import jax
import jax.numpy as jnp
from jax import lax
from jax.experimental import pallas as pl
from jax.experimental.pallas import tpu as pltpu
from jax.experimental.pallas import tpu_sc as plsc

B, N, K, C_IN, D = 4, 2048, 16, 480, 64
KP1 = K + 1
TQ = 256
TM = 512
TA = 256
NW = 32
CHUNK = 128
TBL = 256


def _knn_body(xyz_q_ref, xyz_r_ref, out_ref):
    b = pl.program_id(0)
    q = xyz_q_ref[0]
    r = xyz_r_ref[0]
    sq_q = jnp.sum(q * q, axis=1, keepdims=True)
    sq_r = jnp.sum(r * r, axis=0, keepdims=True)
    inner = jax.lax.dot_general(
        q, r, dimension_numbers=(((1,), (0,)), ((), ())),
        preferred_element_type=jnp.float32,
        precision=jax.lax.Precision.HIGHEST)
    d2 = sq_q + sq_r - 2.0 * inner
    colf = jax.lax.broadcasted_iota(jnp.int32, (TQ, N), 1).astype(jnp.float32)
    bigf = jnp.float32(2.0 * N)
    inf = jnp.float32(jnp.inf)
    picks = []
    for rank in range(KP1):
        m = jnp.min(d2, axis=1, keepdims=True)
        t = jnp.where(d2 <= m, colf, bigf)
        am = jnp.min(t, axis=1, keepdims=True)
        if rank > 0:
            picks.append(am)
        d2 = jnp.where(colf == am, inf, d2)
    idx = jnp.concatenate(picks, axis=1).astype(jnp.int32)
    out_ref[0] = idx + b * N


def _knn(xyz_pm, xyz_cm):
    return pl.pallas_call(
        _knn_body,
        grid=(B, N // TQ),
        in_specs=[
            pl.BlockSpec((1, TQ, 8), lambda b, t: (b, t, 0)),
            pl.BlockSpec((1, 8, N), lambda b, t: (b, 0, 0)),
        ],
        out_specs=pl.BlockSpec((1, TQ, K), lambda b, t: (b, t, 0)),
        out_shape=jax.ShapeDtypeStruct((B, N, K), jnp.int32),
        compiler_params=pltpu.CompilerParams(
            dimension_semantics=("parallel", "parallel")),
    )(xyz_pm, xyz_cm)


def _mlp_body(ft_ref, xyzp_ref, w1_ref, b1_ref, w2_ref, b2_ref, ws_ref,
              bs_ref, wq_ref, bq_ref, wk_ref, bk_ref, wv_ref, bv_ref,
              dw1t_ref, x_ref, q_ref, tbl_ref):
    ftc = ft_ref[0]
    tdot = lambda a, w: jax.lax.dot_general(
        a, w, dimension_numbers=(((0,), (0,)), ((), ())),
        preferred_element_type=jnp.float32)
    h = jnp.maximum(tdot(ftc, w1_ref[...]) + b1_ref[...], 0.0)
    x = (jnp.dot(h, w2_ref[...], preferred_element_type=jnp.float32)
         + b2_ref[...]
         + tdot(ftc, ws_ref[...])
         + bs_ref[...])
    q = jnp.dot(x, wq_ref[...], preferred_element_type=jnp.float32) + bq_ref[...]
    k = jnp.dot(x, wk_ref[...], preferred_element_type=jnp.float32) + bk_ref[...]
    v = jnp.dot(x, wv_ref[...], preferred_element_type=jnp.float32) + bv_ref[...]
    p = jnp.dot(xyzp_ref[0], dw1t_ref[...], preferred_element_type=jnp.float32)
    x_ref[0] = x
    q_ref[0] = q
    tbl_ref[0] = jnp.concatenate(
        [k, v, p, jnp.zeros((TM, D), jnp.float32)], axis=1)


def _mlp_qkv(feature_cm, xyz_p16, weights):
    full = lambda shape: pl.BlockSpec(shape, lambda b, t: tuple(0 for _ in shape))
    wspecs = []
    for w in weights:
        wspecs.append(full(w.shape))
    return pl.pallas_call(
        _mlp_body,
        grid=(B, N // TM),
        in_specs=[
            pl.BlockSpec((1, C_IN, TM), lambda b, t: (b, 0, t)),
            pl.BlockSpec((1, TM, 16), lambda b, t: (b, t, 0)),
        ] + wspecs,
        out_specs=[
            pl.BlockSpec((1, TM, D), lambda b, t: (b, t, 0)),
            pl.BlockSpec((1, TM, D), lambda b, t: (b, t, 0)),
            pl.BlockSpec((1, TM, TBL), lambda b, t: (b, t, 0)),
        ],
        out_shape=[
            jax.ShapeDtypeStruct((B, N, D), jnp.float32),
            jax.ShapeDtypeStruct((B, N, D), jnp.float32),
            jax.ShapeDtypeStruct((B, N, TBL), jnp.float32),
        ],
        compiler_params=pltpu.CompilerParams(
            dimension_semantics=("parallel", "parallel")),
    )(feature_cm, xyz_p16, *weights)


def _sc_gather(table, flat_idx):
    rows = B * N * K
    per_w = rows // NW
    n_chunks = per_w // CHUNK
    mesh = plsc.VectorSubcoreMesh(core_axis_name="c", subcore_axis_name="s")

    @functools.partial(
        pl.kernel,
        mesh=mesh,
        out_type=jax.ShapeDtypeStruct((rows, TBL), jnp.float32),
        scratch_types=[
            pltpu.VMEM((CHUNK,), jnp.int32),
            pltpu.VMEM((CHUNK, TBL), jnp.float32),
            pltpu.SemaphoreType.DMA,
        ],
    )
    def gather_k(table_hbm, idx_hbm, out_hbm, idx_v, rows_v, sem):
        wid = lax.axis_index("s") * 2 + lax.axis_index("c")
        base = wid * per_w

        def body(i, _):
            off = base + i * CHUNK
            pltpu.sync_copy(idx_hbm.at[pl.ds(off, CHUNK)], idx_v)
            pltpu.async_copy(table_hbm.at[idx_v], rows_v, sem).wait()
            pltpu.sync_copy(rows_v, out_hbm.at[pl.ds(off, CHUNK)])
            return ()

        lax.fori_loop(0, n_chunks, body, (), unroll=False)

    return gather_k(table, flat_idx)


def _attn_body(g_ref, q_ref, x_ref, p_ref, db1_ref, dw2_ref,
               db2_ref, gw1_ref, gb1_ref, gw2_ref, gb2_ref, awr_ref, ab_ref,
               m1_ref, mb1_ref, m2_ref, mb2_ref, res_ref, comp_ref):
    M = TA * K
    g = g_ref[0]
    k_l = g[:, :D]
    v_l = g[:, D:2 * D]
    p_l = g[:, 2 * D:3 * D]
    q = q_ref[0]
    x = x_ref[0]
    pq = p_ref[0]

    qb = jnp.broadcast_to(q[:, None, :], (TA, K, D)).reshape(M, D)
    pqb = jnp.broadcast_to(pq[:, None, :], (TA, K, D)).reshape(M, D)

    h = jnp.maximum(pqb - p_l + db1_ref[...], 0.0)
    pos = (jnp.dot(h, dw2_ref[...], preferred_element_type=jnp.float32)
           + db2_ref[...])

    s = qb - k_l + pos
    h2 = jnp.maximum(
        jnp.dot(s, gw1_ref[...], preferred_element_type=jnp.float32)
        + gb1_ref[...], 0.0)
    attn = (jnp.dot(h2, gw2_ref[...], preferred_element_type=jnp.float32)
            + gb2_ref[...])
    value = v_l + pos

    outs = []
    for r in range(4):
        t = (jnp.dot(attn, awr_ref[r], preferred_element_type=jnp.float32)
             + ab_ref[...])
        t = t - jnp.max(t, axis=1, keepdims=True)
        e = jnp.exp(t)
        p = e / jnp.sum(e, axis=1, keepdims=True)
        z = (p * value).reshape(TA, K, D)
        outs.append(jnp.sum(z, axis=1) + x)
    res = jnp.stack(outs, axis=1).reshape(TA * 4, D)
    res_ref[0] = res

    c = jnp.maximum(
        jnp.dot(res, m1_ref[...], preferred_element_type=jnp.float32)
        + mb1_ref[...], 0.0)
    comp_ref[0] = (jnp.dot(c, m2_ref[...], preferred_element_type=jnp.float32)
                   + mb2_ref[...])


def _attn(gathered, q_pm, x_pm, p_pm, weights):
    full = lambda shape: pl.BlockSpec(shape, lambda b, t: tuple(0 for _ in shape))
    wspecs = [full(w.shape) for w in weights]
    return pl.pallas_call(
        _attn_body,
        grid=(B, N // TA),
        in_specs=[
            pl.BlockSpec((1, TA * K, TBL), lambda b, t: (b, t, 0)),
            pl.BlockSpec((1, TA, D), lambda b, t: (b, t, 0)),
            pl.BlockSpec((1, TA, D), lambda b, t: (b, t, 0)),
            pl.BlockSpec((1, TA, D), lambda b, t: (b, t, 0)),
        ] + wspecs,
        out_specs=[
            pl.BlockSpec((1, TA * 4, D), lambda b, t: (b, t, 0)),
            pl.BlockSpec((1, TA * 4, 8), lambda b, t: (b, t, 0)),
        ],
        out_shape=[
            jax.ShapeDtypeStruct((B, 4 * N, D), jnp.float32),
            jax.ShapeDtypeStruct((B, 4 * N, 8), jnp.float32),
        ],
        compiler_params=pltpu.CompilerParams(
            dimension_semantics=("parallel", "parallel")),
    )(gathered, q_pm, x_pm, p_pm, *weights)


def kernel(feature, xyz, bw1, bb1, bw2, bb2, bws, bbs, wq_w, wq_b, wk_w, wk_b,
           wv_w, wv_b, dw1, db1, dg1, dbe1, dw2, db2, gw1, gb1, gg1, gbe1,
           gw2, gb2, aw, ab, m4w1, m4b1, m4w2, m4b2):
    f32 = jnp.float32
    eps = 1e-5

    xyz_pm = jnp.transpose(xyz, (0, 2, 1))
    xyz_p16 = jnp.concatenate(
        [xyz_pm, jnp.zeros((B, N, 13), f32)], axis=2)
    xyz_p8 = xyz_p16[:, :, :8]
    xyz_cm = jnp.concatenate(
        [xyz, jnp.zeros((B, 5, N), f32)], axis=1)

    s1 = dg1 / jnp.sqrt(1.0 + eps)
    sg = gg1 / jnp.sqrt(1.0 + eps)
    dw1t = jnp.concatenate(
        [dw1.T, jnp.zeros((13, D), f32)], axis=0) * s1[None, :]
    db1f = db1 * s1 + dbe1
    gw1t = gw1.T * sg[None, :]
    gb1f = gb1 * sg + gbe1
    awr = jnp.transpose(aw, (2, 0, 1))
    m4w2t = jnp.concatenate(
        [m4w2.T, jnp.zeros((D, 5), f32)], axis=1)
    m4b2p = jnp.concatenate([m4b2, jnp.zeros((5,), f32)])

    idx = _knn(xyz_p8, xyz_cm)
    x_pm, q_pm, table = _mlp_qkv(
        feature, xyz_p16,
        [bw1.T, bb1[None, :], bw2.T, bb2[None, :], bws.T, bbs[None, :],
         wq_w.T, wq_b[None, :], wk_w.T, wk_b[None, :], wv_w.T, wv_b[None, :],
         dw1t])
    p_pm = table[:, :, 2 * D:3 * D]

    gathered = _sc_gather(table.reshape(B * N, TBL), idx.reshape(B * N * K))
    gathered = gathered.reshape(B, N * K, TBL)

    res_pm, comp_pm = _attn(
        gathered, q_pm, x_pm, p_pm,
        [db1f[None, :], dw2.T, db2[None, :], gw1t, gb1f[None, :],
         gw2.T, gb2[None, :], awr, ab[None, :], m4w1.T, m4b1[None, :],
         m4w2t, m4b2p[None, :]])

    res = jnp.transpose(res_pm, (0, 2, 1))
    completion = jnp.transpose(comp_pm[:, :, :3], (0, 2, 1))
    return (completion, res)


# speedup vs baseline: 355.7521x; 1.0687x over previous
import functools

import jax
import jax.numpy as jnp
from jax import lax
from jax.experimental import pallas as pl
from jax.experimental.pallas import tpu as pltpu
from jax.experimental.pallas import tpu_sc as plsc

B, N, K, C_IN, D = 4, 2048, 16, 480, 64
KP1 = K + 1
TQ = 512
TM = 512
TA = 256
NW = 32
CHUNK = 128
TBL = 256


def _knn_body(xyz_q_ref, xyz_r_ref, out_ref):
    b = pl.program_id(0)
    t_id = pl.program_id(1)
    q = xyz_q_ref[0]
    r = xyz_r_ref[0]
    sq_q = jnp.sum(q * q, axis=1, keepdims=True)
    sq_r = jnp.sum(r * r, axis=0, keepdims=True)
    inner = jax.lax.dot_general(
        q, r, dimension_numbers=(((1,), (0,)), ((), ())),
        preferred_element_type=jnp.float32,
        precision=jax.lax.Precision.HIGHEST)
    d2 = sq_q + sq_r - 2.0 * inner
    colf = jax.lax.broadcasted_iota(jnp.int32, (TQ, N), 1).astype(jnp.float32)
    bigf = jnp.float32(2.0 * N)
    inf = jnp.float32(jnp.inf)
    rowf = (jax.lax.broadcasted_iota(jnp.int32, (TQ, 1), 0)
            + t_id * TQ).astype(jnp.float32)
    d2 = jnp.where(colf == rowf, inf, d2)
    picks = []
    for _ in range(K):
        m = jnp.min(d2, axis=1, keepdims=True)
        t = jnp.where(d2 <= m, colf, bigf)
        am = jnp.min(t, axis=1, keepdims=True)
        picks.append(am)
        d2 = jnp.where(colf == am, inf, d2)
    idx = jnp.concatenate(picks, axis=1).astype(jnp.int32)
    out_ref[0] = idx + b * N


def _knn(xyz_pm, xyz_cm):
    return pl.pallas_call(
        _knn_body,
        grid=(B, N // TQ),
        in_specs=[
            pl.BlockSpec((1, TQ, 8), lambda b, t: (b, t, 0)),
            pl.BlockSpec((1, 8, N), lambda b, t: (b, 0, 0)),
        ],
        out_specs=pl.BlockSpec((1, TQ, K), lambda b, t: (b, t, 0)),
        out_shape=jax.ShapeDtypeStruct((B, N, K), jnp.int32),
        compiler_params=pltpu.CompilerParams(
            dimension_semantics=("parallel", "parallel")),
    )(xyz_pm, xyz_cm)


def _mlp_body(ft_ref, xyzp_ref, w1_ref, b1_ref, w2_ref, b2_ref, ws_ref,
              bs_ref, wq_ref, bq_ref, wk_ref, bk_ref, wv_ref, bv_ref,
              dw1t_ref, x_ref, q_ref, tbl_ref):
    ftc = ft_ref[0]
    tdot = lambda a, w: jax.lax.dot_general(
        a, w, dimension_numbers=(((0,), (0,)), ((), ())),
        preferred_element_type=jnp.float32)
    h = jnp.maximum(tdot(ftc, w1_ref[...]) + b1_ref[...], 0.0)
    x = (jnp.dot(h, w2_ref[...], preferred_element_type=jnp.float32)
         + b2_ref[...]
         + tdot(ftc, ws_ref[...])
         + bs_ref[...])
    q = jnp.dot(x, wq_ref[...], preferred_element_type=jnp.float32) + bq_ref[...]
    k = jnp.dot(x, wk_ref[...], preferred_element_type=jnp.float32) + bk_ref[...]
    v = jnp.dot(x, wv_ref[...], preferred_element_type=jnp.float32) + bv_ref[...]
    p = jnp.dot(xyzp_ref[0], dw1t_ref[...], preferred_element_type=jnp.float32)
    x_ref[0] = x
    q_ref[0] = q
    tbl_ref[0] = jnp.concatenate(
        [k, v, p, jnp.zeros((TM, D), jnp.float32)], axis=1)


def _mlp_qkv(feature_cm, xyz_p16, weights):
    full = lambda shape: pl.BlockSpec(shape, lambda b, t: tuple(0 for _ in shape))
    wspecs = []
    for w in weights:
        wspecs.append(full(w.shape))
    return pl.pallas_call(
        _mlp_body,
        grid=(B, N // TM),
        in_specs=[
            pl.BlockSpec((1, C_IN, TM), lambda b, t: (b, 0, t)),
            pl.BlockSpec((1, TM, 16), lambda b, t: (b, t, 0)),
        ] + wspecs,
        out_specs=[
            pl.BlockSpec((1, TM, D), lambda b, t: (b, t, 0)),
            pl.BlockSpec((1, TM, D), lambda b, t: (b, t, 0)),
            pl.BlockSpec((1, TM, TBL), lambda b, t: (b, t, 0)),
        ],
        out_shape=[
            jax.ShapeDtypeStruct((B, N, D), jnp.float32),
            jax.ShapeDtypeStruct((B, N, D), jnp.float32),
            jax.ShapeDtypeStruct((B, N, TBL), jnp.float32),
        ],
        compiler_params=pltpu.CompilerParams(
            dimension_semantics=("parallel", "parallel")),
    )(feature_cm, xyz_p16, *weights)


def _sc_gather(table, flat_idx2d):
    rows = B * N * K
    per_w = rows // NW
    n_chunks = per_w // CHUNK
    mesh = plsc.VectorSubcoreMesh(core_axis_name="c", subcore_axis_name="s")

    @functools.partial(
        pl.kernel,
        mesh=mesh,
        out_type=jax.ShapeDtypeStruct((rows, TBL), jnp.float32),
        scratch_types=[
            pltpu.VMEM((n_chunks, CHUNK), jnp.int32),
            pltpu.VMEM((CHUNK, TBL), jnp.float32),
            pltpu.VMEM((CHUNK, TBL), jnp.float32),
            pltpu.SemaphoreType.DMA,
            pltpu.SemaphoreType.DMA,
            pltpu.SemaphoreType.DMA,
            pltpu.SemaphoreType.DMA,
        ],
    )
    def gather_k(table_hbm, idx_hbm, out_hbm, idx_v, rows_a, rows_b,
                 g_sem_a, g_sem_b, s_sem_a, s_sem_b):
        wid = lax.axis_index("s") * 2 + lax.axis_index("c")
        base = wid * per_w
        pltpu.sync_copy(idx_hbm.at[pl.ds(wid * n_chunks, n_chunks)], idx_v)

        def body(i, _):
            hs = []
            for b, rows_v, g_sem in ((0, rows_a, g_sem_a), (1, rows_b, g_sem_b)):
                c = 2 * i + b
                hs.append(pltpu.async_copy(
                    table_hbm.at[idx_v.at[c]], rows_v, g_sem))
            ss = []
            for b, rows_v, g_sem, s_sem in (
                    (0, rows_a, g_sem_a, s_sem_a),
                    (1, rows_b, g_sem_b, s_sem_b)):
                c = 2 * i + b
                hs[b].wait()
                ss.append(pltpu.async_copy(
                    rows_v, out_hbm.at[pl.ds(base + c * CHUNK, CHUNK)], s_sem))
            for h in ss:
                h.wait()
            return ()

        lax.fori_loop(0, n_chunks // 2, body, (), unroll=False)

    return gather_k(table, flat_idx2d)


def _attn_body(g_ref, q_ref, x_ref, p_ref, db1_ref, dw2_ref,
               db2_ref, gw1_ref, gb1_ref, gw2_ref, gb2_ref, awr_ref, ab_ref,
               m1_ref, mb1_ref, m2_ref, mb2_ref, res_ref, comp_ref):
    M = TA * K
    g = g_ref[0]
    k_l = g[:, :D]
    v_l = g[:, D:2 * D]
    p_l = g[:, 2 * D:3 * D]
    q = q_ref[0]
    x = x_ref[0]
    pq = p_ref[0]

    qb = jnp.broadcast_to(q[:, None, :], (TA, K, D)).reshape(M, D)
    pqb = jnp.broadcast_to(pq[:, None, :], (TA, K, D)).reshape(M, D)

    h = jnp.maximum(pqb - p_l + db1_ref[...], 0.0)
    pos = (jnp.dot(h, dw2_ref[...], preferred_element_type=jnp.float32)
           + db2_ref[...])

    s = qb - k_l + pos
    h2 = jnp.maximum(
        jnp.dot(s, gw1_ref[...], preferred_element_type=jnp.float32)
        + gb1_ref[...], 0.0)
    attn = (jnp.dot(h2, gw2_ref[...], preferred_element_type=jnp.float32)
            + gb2_ref[...])
    value = v_l + pos

    outs = []
    for r in range(4):
        t = (jnp.dot(attn, awr_ref[r], preferred_element_type=jnp.float32)
             + ab_ref[...])
        t = t - jnp.max(t, axis=1, keepdims=True)
        e = jnp.exp(t)
        p = e / jnp.sum(e, axis=1, keepdims=True)
        z = (p * value).reshape(TA, K, D)
        outs.append(jnp.sum(z, axis=1) + x)
    res = jnp.stack(outs, axis=1).reshape(TA * 4, D)
    res_ref[0] = res

    c = jnp.maximum(
        jnp.dot(res, m1_ref[...], preferred_element_type=jnp.float32)
        + mb1_ref[...], 0.0)
    comp_ref[0] = (jnp.dot(c, m2_ref[...], preferred_element_type=jnp.float32)
                   + mb2_ref[...])


def _attn(gathered, q_pm, x_pm, p_pm, weights):
    full = lambda shape: pl.BlockSpec(shape, lambda b, t: tuple(0 for _ in shape))
    wspecs = [full(w.shape) for w in weights]
    return pl.pallas_call(
        _attn_body,
        grid=(B, N // TA),
        in_specs=[
            pl.BlockSpec((1, TA * K, TBL), lambda b, t: (b, t, 0)),
            pl.BlockSpec((1, TA, D), lambda b, t: (b, t, 0)),
            pl.BlockSpec((1, TA, D), lambda b, t: (b, t, 0)),
            pl.BlockSpec((1, TA, D), lambda b, t: (b, t, 0)),
        ] + wspecs,
        out_specs=[
            pl.BlockSpec((1, TA * 4, D), lambda b, t: (b, t, 0)),
            pl.BlockSpec((1, TA * 4, 8), lambda b, t: (b, t, 0)),
        ],
        out_shape=[
            jax.ShapeDtypeStruct((B, 4 * N, D), jnp.float32),
            jax.ShapeDtypeStruct((B, 4 * N, 8), jnp.float32),
        ],
        compiler_params=pltpu.CompilerParams(
            dimension_semantics=("parallel", "parallel")),
    )(gathered, q_pm, x_pm, p_pm, *weights)


def kernel(feature, xyz, bw1, bb1, bw2, bb2, bws, bbs, wq_w, wq_b, wk_w, wk_b,
           wv_w, wv_b, dw1, db1, dg1, dbe1, dw2, db2, gw1, gb1, gg1, gbe1,
           gw2, gb2, aw, ab, m4w1, m4b1, m4w2, m4b2):
    f32 = jnp.float32
    eps = 1e-5

    xyz_pm = jnp.transpose(xyz, (0, 2, 1))
    xyz_p16 = jnp.concatenate(
        [xyz_pm, jnp.zeros((B, N, 13), f32)], axis=2)
    xyz_p8 = xyz_p16[:, :, :8]
    xyz_cm = jnp.concatenate(
        [xyz, jnp.zeros((B, 5, N), f32)], axis=1)

    s1 = dg1 / jnp.sqrt(1.0 + eps)
    sg = gg1 / jnp.sqrt(1.0 + eps)
    dw1t = jnp.concatenate(
        [dw1.T, jnp.zeros((13, D), f32)], axis=0) * s1[None, :]
    db1f = db1 * s1 + dbe1
    gw1t = gw1.T * sg[None, :]
    gb1f = gb1 * sg + gbe1
    awr = jnp.transpose(aw, (2, 0, 1))
    m4w2t = jnp.concatenate(
        [m4w2.T, jnp.zeros((D, 5), f32)], axis=1)
    m4b2p = jnp.concatenate([m4b2, jnp.zeros((5,), f32)])

    idx = _knn(xyz_p8, xyz_cm)
    x_pm, q_pm, table = _mlp_qkv(
        feature, xyz_p16,
        [bw1.T, bb1[None, :], bw2.T, bb2[None, :], bws.T, bbs[None, :],
         wq_w.T, wq_b[None, :], wk_w.T, wk_b[None, :], wv_w.T, wv_b[None, :],
         dw1t])
    p_pm = table[:, :, 2 * D:3 * D]

    gathered = _sc_gather(table.reshape(B * N, TBL),
                          idx.reshape(B * N * K // CHUNK, CHUNK))
    gathered = gathered.reshape(B, N * K, TBL)

    res_pm, comp_pm = _attn(
        gathered, q_pm, x_pm, p_pm,
        [db1f[None, :], dw2.T, db2[None, :], gw1t, gb1f[None, :],
         gw2.T, gb2[None, :], awr, ab[None, :], m4w1.T, m4b1[None, :],
         m4w2t, m4b2p[None, :]])

    res = jnp.transpose(res_pm, (0, 2, 1))
    completion = jnp.transpose(comp_pm[:, :, :3], (0, 2, 1))
    return (completion, res)


# speedup vs baseline: 360.3423x; 1.0129x over previous
import functools

import jax
import jax.numpy as jnp
from jax import lax
from jax.experimental import pallas as pl
from jax.experimental.pallas import tpu as pltpu
from jax.experimental.pallas import tpu_sc as plsc

B, N, K, C_IN, D = 4, 2048, 16, 480, 64
KP1 = K + 1
TQ = 512
TM = 512
TA = 256
NW = 32
CHUNK = 128
TBL = 256


def _knn_body(xyz_q_ref, xyz_r_ref, ft_ref, xyzp_ref, w1_ref, b1_ref,
              w2_ref, b2_ref, ws_ref, bs_ref, wq_ref, bq_ref, wk_ref,
              bk_ref, wv_ref, bv_ref, dw1t_ref, out_ref, x_ref, q_ref,
              tbl_ref, p_ref):
    b = pl.program_id(0)
    t_id = pl.program_id(1)

    ftc = ft_ref[0]
    tdot = lambda a, w: jax.lax.dot_general(
        a, w, dimension_numbers=(((0,), (0,)), ((), ())),
        preferred_element_type=jnp.float32)
    h = jnp.maximum(tdot(ftc, w1_ref[...]) + b1_ref[...], 0.0)
    x = (jnp.dot(h, w2_ref[...], preferred_element_type=jnp.float32)
         + b2_ref[...]
         + tdot(ftc, ws_ref[...])
         + bs_ref[...])
    xq = jnp.dot(x, wq_ref[...], preferred_element_type=jnp.float32) + bq_ref[...]
    xk = jnp.dot(x, wk_ref[...], preferred_element_type=jnp.float32) + bk_ref[...]
    xv = jnp.dot(x, wv_ref[...], preferred_element_type=jnp.float32) + bv_ref[...]
    p = jnp.dot(xyzp_ref[0], dw1t_ref[...], preferred_element_type=jnp.float32)
    x_ref[0] = x
    q_ref[0] = xq
    p_ref[0] = p
    tbl_ref[0] = jnp.concatenate(
        [xk, xv, p, jnp.zeros((TQ, D), jnp.float32)], axis=1)

    q = xyz_q_ref[0]
    r = xyz_r_ref[0]
    sq_q = jnp.sum(q * q, axis=1, keepdims=True)
    sq_r = jnp.sum(r * r, axis=0, keepdims=True)
    inner = jax.lax.dot_general(
        q, r, dimension_numbers=(((1,), (0,)), ((), ())),
        preferred_element_type=jnp.float32,
        precision=jax.lax.Precision.HIGHEST)
    d2 = sq_q + sq_r - 2.0 * inner
    colf = jax.lax.broadcasted_iota(jnp.int32, (TQ, N), 1).astype(jnp.float32)
    bigf = jnp.float32(2.0 * N)
    inf = jnp.float32(jnp.inf)
    rowf = (jax.lax.broadcasted_iota(jnp.int32, (TQ, 1), 0)
            + t_id * TQ).astype(jnp.float32)
    d2 = jnp.where(colf == rowf, inf, d2)
    picks = []
    for _ in range(K):
        m = jnp.min(d2, axis=1, keepdims=True)
        t = jnp.where(d2 <= m, colf, bigf)
        am = jnp.min(t, axis=1, keepdims=True)
        picks.append(am)
        d2 = jnp.where(colf == am, inf, d2)
    idx = jnp.concatenate(picks, axis=1).astype(jnp.int32)
    out_ref[0] = idx + b * N


def _knn_mlp(xyz_pm, xyz_cm, feature_cm, xyz_p16, weights):
    full = lambda shape: pl.BlockSpec(shape, lambda b, t: tuple(0 for _ in shape))
    wspecs = [full(w.shape) for w in weights]
    return pl.pallas_call(
        _knn_body,
        grid=(B, N // TQ),
        in_specs=[
            pl.BlockSpec((1, TQ, 8), lambda b, t: (b, t, 0)),
            pl.BlockSpec((1, 8, N), lambda b, t: (b, 0, 0)),
            pl.BlockSpec((1, C_IN, TQ), lambda b, t: (b, 0, t)),
            pl.BlockSpec((1, TQ, 16), lambda b, t: (b, t, 0)),
        ] + wspecs,
        out_specs=[
            pl.BlockSpec((1, TQ, K), lambda b, t: (b, t, 0)),
            pl.BlockSpec((1, TQ, D), lambda b, t: (b, t, 0)),
            pl.BlockSpec((1, TQ, D), lambda b, t: (b, t, 0)),
            pl.BlockSpec((1, TQ, TBL), lambda b, t: (b, t, 0)),
            pl.BlockSpec((1, TQ, D), lambda b, t: (b, t, 0)),
        ],
        out_shape=[
            jax.ShapeDtypeStruct((B, N, K), jnp.int32),
            jax.ShapeDtypeStruct((B, N, D), jnp.float32),
            jax.ShapeDtypeStruct((B, N, D), jnp.float32),
            jax.ShapeDtypeStruct((B, N, TBL), jnp.float32),
            jax.ShapeDtypeStruct((B, N, D), jnp.float32),
        ],
        compiler_params=pltpu.CompilerParams(
            dimension_semantics=("parallel", "parallel")),
    )(xyz_pm, xyz_cm, feature_cm, xyz_p16, *weights)


def _sc_gather(table, flat_idx2d):
    rows = B * N * K
    per_w = rows // NW
    n_chunks = per_w // CHUNK
    mesh = plsc.VectorSubcoreMesh(core_axis_name="c", subcore_axis_name="s")

    @functools.partial(
        pl.kernel,
        mesh=mesh,
        out_type=jax.ShapeDtypeStruct((rows, TBL), jnp.float32),
        scratch_types=[
            pltpu.VMEM((n_chunks, CHUNK), jnp.int32),
            pltpu.VMEM((CHUNK, TBL), jnp.float32),
            pltpu.VMEM((CHUNK, TBL), jnp.float32),
            pltpu.SemaphoreType.DMA,
            pltpu.SemaphoreType.DMA,
            pltpu.SemaphoreType.DMA,
            pltpu.SemaphoreType.DMA,
        ],
    )
    def gather_k(table_hbm, idx_hbm, out_hbm, idx_v, rows_a, rows_b,
                 g_sem_a, g_sem_b, s_sem_a, s_sem_b):
        wid = lax.axis_index("s") * 2 + lax.axis_index("c")
        base = wid * per_w
        pltpu.sync_copy(idx_hbm.at[pl.ds(wid * n_chunks, n_chunks)], idx_v)

        def body(i, _):
            hs = []
            for b, rows_v, g_sem in ((0, rows_a, g_sem_a), (1, rows_b, g_sem_b)):
                c = 2 * i + b
                hs.append(pltpu.async_copy(
                    table_hbm.at[idx_v.at[c]], rows_v, g_sem))
            ss = []
            for b, rows_v, g_sem, s_sem in (
                    (0, rows_a, g_sem_a, s_sem_a),
                    (1, rows_b, g_sem_b, s_sem_b)):
                c = 2 * i + b
                hs[b].wait()
                ss.append(pltpu.async_copy(
                    rows_v, out_hbm.at[pl.ds(base + c * CHUNK, CHUNK)], s_sem))
            for h in ss:
                h.wait()
            return ()

        lax.fori_loop(0, n_chunks // 2, body, (), unroll=False)

    return gather_k(table, flat_idx2d)


def _attn_body(g_ref, q_ref, x_ref, p_ref, db1_ref, dw2_ref,
               db2_ref, gw1_ref, gb1_ref, gw2_ref, gb2_ref, awr_ref, ab_ref,
               m1_ref, mb1_ref, m2_ref, mb2_ref, res_ref, comp_ref):
    M = TA * K
    g = g_ref[0]
    k_l = g[:, :D]
    v_l = g[:, D:2 * D]
    p_l = g[:, 2 * D:3 * D]
    q = q_ref[0]
    x = x_ref[0]
    pq = p_ref[0]

    qb = jnp.broadcast_to(q[:, None, :], (TA, K, D)).reshape(M, D)
    pqb = jnp.broadcast_to(pq[:, None, :], (TA, K, D)).reshape(M, D)

    h = jnp.maximum(pqb - p_l + db1_ref[...], 0.0)
    pos = (jnp.dot(h, dw2_ref[...], preferred_element_type=jnp.float32)
           + db2_ref[...])

    s = qb - k_l + pos
    h2 = jnp.maximum(
        jnp.dot(s, gw1_ref[...], preferred_element_type=jnp.float32)
        + gb1_ref[...], 0.0)
    attn = (jnp.dot(h2, gw2_ref[...], preferred_element_type=jnp.float32)
            + gb2_ref[...])
    value = v_l + pos

    outs = []
    for r in range(4):
        t = (jnp.dot(attn, awr_ref[r], preferred_element_type=jnp.float32)
             + ab_ref[...])
        t = t - jnp.max(t, axis=1, keepdims=True)
        e = jnp.exp(t)
        p = e / jnp.sum(e, axis=1, keepdims=True)
        z = (p * value).reshape(TA, K, D)
        outs.append(jnp.sum(z, axis=1) + x)
    res = jnp.stack(outs, axis=1).reshape(TA * 4, D)
    res_ref[0] = res

    c = jnp.maximum(
        jnp.dot(res, m1_ref[...], preferred_element_type=jnp.float32)
        + mb1_ref[...], 0.0)
    comp_ref[0] = (jnp.dot(c, m2_ref[...], preferred_element_type=jnp.float32)
                   + mb2_ref[...])


def _attn(gathered, q_pm, x_pm, p_pm, weights):
    full = lambda shape: pl.BlockSpec(shape, lambda b, t: tuple(0 for _ in shape))
    wspecs = [full(w.shape) for w in weights]
    return pl.pallas_call(
        _attn_body,
        grid=(B, N // TA),
        in_specs=[
            pl.BlockSpec((1, TA * K, TBL), lambda b, t: (b, t, 0)),
            pl.BlockSpec((1, TA, D), lambda b, t: (b, t, 0)),
            pl.BlockSpec((1, TA, D), lambda b, t: (b, t, 0)),
            pl.BlockSpec((1, TA, D), lambda b, t: (b, t, 0)),
        ] + wspecs,
        out_specs=[
            pl.BlockSpec((1, TA * 4, D), lambda b, t: (b, t, 0)),
            pl.BlockSpec((1, TA * 4, 8), lambda b, t: (b, t, 0)),
        ],
        out_shape=[
            jax.ShapeDtypeStruct((B, 4 * N, D), jnp.float32),
            jax.ShapeDtypeStruct((B, 4 * N, 8), jnp.float32),
        ],
        compiler_params=pltpu.CompilerParams(
            dimension_semantics=("parallel", "parallel")),
    )(gathered, q_pm, x_pm, p_pm, *weights)


def kernel(feature, xyz, bw1, bb1, bw2, bb2, bws, bbs, wq_w, wq_b, wk_w, wk_b,
           wv_w, wv_b, dw1, db1, dg1, dbe1, dw2, db2, gw1, gb1, gg1, gbe1,
           gw2, gb2, aw, ab, m4w1, m4b1, m4w2, m4b2):
    f32 = jnp.float32
    eps = 1e-5

    xyz_pm = jnp.transpose(xyz, (0, 2, 1))
    xyz_p16 = jnp.concatenate(
        [xyz_pm, jnp.zeros((B, N, 13), f32)], axis=2)
    xyz_p8 = xyz_p16[:, :, :8]
    xyz_cm = jnp.concatenate(
        [xyz, jnp.zeros((B, 5, N), f32)], axis=1)

    s1 = dg1 / jnp.sqrt(1.0 + eps)
    sg = gg1 / jnp.sqrt(1.0 + eps)
    dw1t = jnp.concatenate(
        [dw1.T, jnp.zeros((13, D), f32)], axis=0) * s1[None, :]
    db1f = db1 * s1 + dbe1
    gw1t = gw1.T * sg[None, :]
    gb1f = gb1 * sg + gbe1
    awr = jnp.transpose(aw, (2, 0, 1))
    m4w2t = jnp.concatenate(
        [m4w2.T, jnp.zeros((D, 5), f32)], axis=1)
    m4b2p = jnp.concatenate([m4b2, jnp.zeros((5,), f32)])

    idx, x_pm, q_pm, table, p_pm = _knn_mlp(
        xyz_p8, xyz_cm, feature, xyz_p16,
        [bw1.T, bb1[None, :], bw2.T, bb2[None, :], bws.T, bbs[None, :],
         wq_w.T, wq_b[None, :], wk_w.T, wk_b[None, :], wv_w.T, wv_b[None, :],
         dw1t])

    gathered = _sc_gather(table.reshape(B * N, TBL),
                          idx.reshape(B * N * K // CHUNK, CHUNK))
    gathered = gathered.reshape(B, N * K, TBL)

    res_pm, comp_pm = _attn(
        gathered, q_pm, x_pm, p_pm,
        [db1f[None, :], dw2.T, db2[None, :], gw1t, gb1f[None, :],
         gw2.T, gb2[None, :], awr, ab[None, :], m4w1.T, m4b1[None, :],
         m4w2t, m4b2p[None, :]])

    res = jnp.transpose(res_pm, (0, 2, 1))
    completion = jnp.transpose(comp_pm[:, :, :3], (0, 2, 1))
    return (completion, res)


# speedup vs baseline: 393.5074x; 1.0920x over previous
import functools

import jax
import jax.numpy as jnp
from jax import lax
from jax.experimental import pallas as pl
from jax.experimental.pallas import tpu as pltpu
from jax.experimental.pallas import tpu_sc as plsc

B, N, K, C_IN, D = 4, 2048, 16, 480, 64
KP1 = K + 1
TQ = 512
TM = 512
TA = 256
NW = 32
CHUNK = 128
TBL = 256


def _knn_body(xyz_q_ref, xyz_r_ref, ft_ref, xyzp_ref, w1_ref, b1_ref,
              w2_ref, b2_ref, ws_ref, bs_ref, wq_ref, bq_ref, wk_ref,
              bk_ref, wv_ref, bv_ref, dw1t_ref, out_ref, x_ref, q_ref,
              tbl_ref, p_ref):
    b = pl.program_id(0)
    t_id = pl.program_id(1)

    ftc = ft_ref[0]
    tdot = lambda a, w: jax.lax.dot_general(
        a, w, dimension_numbers=(((0,), (0,)), ((), ())),
        preferred_element_type=jnp.float32)
    h = jnp.maximum(tdot(ftc, w1_ref[...]) + b1_ref[...], 0.0)
    x = (jnp.dot(h, w2_ref[...], preferred_element_type=jnp.float32)
         + b2_ref[...]
         + tdot(ftc, ws_ref[...])
         + bs_ref[...])
    xq = jnp.dot(x, wq_ref[...], preferred_element_type=jnp.float32) + bq_ref[...]
    xk = jnp.dot(x, wk_ref[...], preferred_element_type=jnp.float32) + bk_ref[...]
    xv = jnp.dot(x, wv_ref[...], preferred_element_type=jnp.float32) + bv_ref[...]
    p = jnp.dot(xyzp_ref[0], dw1t_ref[...], preferred_element_type=jnp.float32)
    x_ref[0] = x
    q_ref[0] = xq
    p_ref[0] = p
    tbl_ref[0] = jnp.concatenate(
        [xk, xv, p, jnp.zeros((TQ, D), jnp.float32)], axis=1)

    q = xyz_q_ref[0]
    r = xyz_r_ref[0]
    sq_q = jnp.sum(q * q, axis=1, keepdims=True)
    sq_r = jnp.sum(r * r, axis=0, keepdims=True)
    inner = jax.lax.dot_general(
        q, r, dimension_numbers=(((1,), (0,)), ((), ())),
        preferred_element_type=jnp.float32,
        precision=jax.lax.Precision.HIGHEST)
    d2 = sq_q + sq_r - 2.0 * inner
    colf = jax.lax.broadcasted_iota(jnp.int32, (TQ, N), 1).astype(jnp.float32)
    bigf = jnp.float32(2.0 * N)
    inf = jnp.float32(jnp.inf)
    rowf = (jax.lax.broadcasted_iota(jnp.int32, (TQ, 1), 0)
            + t_id * TQ).astype(jnp.float32)
    d2 = jnp.where(colf == rowf, inf, d2)
    picks = []
    for _ in range(K):
        m = jnp.min(d2, axis=1, keepdims=True)
        t = jnp.where(d2 <= m, colf, bigf)
        am = jnp.min(t, axis=1, keepdims=True)
        picks.append(am)
        d2 = jnp.where(colf == am, inf, d2)
    idx = jnp.concatenate(picks, axis=1).astype(jnp.int32)
    out_ref[0] = idx + b * N


def _knn_mlp(xyz_pm, xyz_cm, feature_cm, xyz_p16, weights, bsz):
    full = lambda shape: pl.BlockSpec(shape, lambda b, t: tuple(0 for _ in shape))
    wspecs = [full(w.shape) for w in weights]
    return pl.pallas_call(
        _knn_body,
        grid=(bsz, N // TQ),
        in_specs=[
            pl.BlockSpec((1, TQ, 8), lambda b, t: (b, t, 0)),
            pl.BlockSpec((1, 8, N), lambda b, t: (b, 0, 0)),
            pl.BlockSpec((1, C_IN, TQ), lambda b, t: (b, 0, t)),
            pl.BlockSpec((1, TQ, 16), lambda b, t: (b, t, 0)),
        ] + wspecs,
        out_specs=[
            pl.BlockSpec((1, TQ, K), lambda b, t: (b, t, 0)),
            pl.BlockSpec((1, TQ, D), lambda b, t: (b, t, 0)),
            pl.BlockSpec((1, TQ, D), lambda b, t: (b, t, 0)),
            pl.BlockSpec((1, TQ, TBL), lambda b, t: (b, t, 0)),
            pl.BlockSpec((1, TQ, D), lambda b, t: (b, t, 0)),
        ],
        out_shape=[
            jax.ShapeDtypeStruct((bsz, N, K), jnp.int32),
            jax.ShapeDtypeStruct((bsz, N, D), jnp.float32),
            jax.ShapeDtypeStruct((bsz, N, D), jnp.float32),
            jax.ShapeDtypeStruct((bsz, N, TBL), jnp.float32),
            jax.ShapeDtypeStruct((bsz, N, D), jnp.float32),
        ],
        compiler_params=pltpu.CompilerParams(
            dimension_semantics=("parallel", "parallel")),
    )(xyz_pm, xyz_cm, feature_cm, xyz_p16, *weights)


def _sc_gather(table, flat_idx2d, bsz):
    rows = bsz * N * K
    per_w = rows // NW
    n_chunks = per_w // CHUNK
    mesh = plsc.VectorSubcoreMesh(core_axis_name="c", subcore_axis_name="s")

    @functools.partial(
        pl.kernel,
        mesh=mesh,
        out_type=jax.ShapeDtypeStruct((rows, TBL), jnp.float32),
        scratch_types=[
            pltpu.VMEM((n_chunks, CHUNK), jnp.int32),
            pltpu.VMEM((CHUNK, TBL), jnp.float32),
            pltpu.VMEM((CHUNK, TBL), jnp.float32),
            pltpu.SemaphoreType.DMA,
            pltpu.SemaphoreType.DMA,
            pltpu.SemaphoreType.DMA,
            pltpu.SemaphoreType.DMA,
        ],
    )
    def gather_k(table_hbm, idx_hbm, out_hbm, idx_v, rows_a, rows_b,
                 g_sem_a, g_sem_b, s_sem_a, s_sem_b):
        wid = lax.axis_index("s") * 2 + lax.axis_index("c")
        base = wid * per_w
        pltpu.sync_copy(idx_hbm.at[pl.ds(wid * n_chunks, n_chunks)], idx_v)

        def body(i, _):
            hs = []
            for b, rows_v, g_sem in ((0, rows_a, g_sem_a), (1, rows_b, g_sem_b)):
                c = 2 * i + b
                hs.append(pltpu.async_copy(
                    table_hbm.at[idx_v.at[c]], rows_v, g_sem))
            ss = []
            for b, rows_v, g_sem, s_sem in (
                    (0, rows_a, g_sem_a, s_sem_a),
                    (1, rows_b, g_sem_b, s_sem_b)):
                c = 2 * i + b
                hs[b].wait()
                ss.append(pltpu.async_copy(
                    rows_v, out_hbm.at[pl.ds(base + c * CHUNK, CHUNK)], s_sem))
            for h in ss:
                h.wait()
            return ()

        lax.fori_loop(0, n_chunks // 2, body, (), unroll=False)

    return gather_k(table, flat_idx2d)


def _attn_body(g_ref, q_ref, x_ref, p_ref, db1_ref, dw2_ref,
               db2_ref, gw1_ref, gb1_ref, gw2_ref, gb2_ref, awr_ref, ab_ref,
               m1_ref, mb1_ref, m2_ref, mb2_ref, res_ref, comp_ref):
    M = TA * K
    g = g_ref[0]
    k_l = g[:, :D]
    v_l = g[:, D:2 * D]
    p_l = g[:, 2 * D:3 * D]
    q = q_ref[0]
    x = x_ref[0]
    pq = p_ref[0]

    qb = jnp.broadcast_to(q[:, None, :], (TA, K, D)).reshape(M, D)
    pqb = jnp.broadcast_to(pq[:, None, :], (TA, K, D)).reshape(M, D)

    h = jnp.maximum(pqb - p_l + db1_ref[...], 0.0)
    pos = (jnp.dot(h, dw2_ref[...], preferred_element_type=jnp.float32)
           + db2_ref[...])

    s = qb - k_l + pos
    h2 = jnp.maximum(
        jnp.dot(s, gw1_ref[...], preferred_element_type=jnp.float32)
        + gb1_ref[...], 0.0)
    attn = (jnp.dot(h2, gw2_ref[...], preferred_element_type=jnp.float32)
            + gb2_ref[...])
    value = v_l + pos

    outs = []
    for r in range(4):
        t = (jnp.dot(attn, awr_ref[r], preferred_element_type=jnp.float32)
             + ab_ref[...])
        t = t - jnp.max(t, axis=1, keepdims=True)
        e = jnp.exp(t)
        p = e / jnp.sum(e, axis=1, keepdims=True)
        z = (p * value).reshape(TA, K, D)
        outs.append(jnp.sum(z, axis=1) + x)
    res = jnp.stack(outs, axis=1).reshape(TA * 4, D)
    res_ref[0] = res

    c = jnp.maximum(
        jnp.dot(res, m1_ref[...], preferred_element_type=jnp.float32)
        + mb1_ref[...], 0.0)
    comp_ref[0] = (jnp.dot(c, m2_ref[...], preferred_element_type=jnp.float32)
                   + mb2_ref[...])


def _attn(gathered, q_pm, x_pm, p_pm, weights, bsz):
    full = lambda shape: pl.BlockSpec(shape, lambda b, t: tuple(0 for _ in shape))
    wspecs = [full(w.shape) for w in weights]
    return pl.pallas_call(
        _attn_body,
        grid=(bsz, N // TA),
        in_specs=[
            pl.BlockSpec((1, TA * K, TBL), lambda b, t: (b, t, 0)),
            pl.BlockSpec((1, TA, D), lambda b, t: (b, t, 0)),
            pl.BlockSpec((1, TA, D), lambda b, t: (b, t, 0)),
            pl.BlockSpec((1, TA, D), lambda b, t: (b, t, 0)),
        ] + wspecs,
        out_specs=[
            pl.BlockSpec((1, TA * 4, D), lambda b, t: (b, t, 0)),
            pl.BlockSpec((1, TA * 4, 8), lambda b, t: (b, t, 0)),
        ],
        out_shape=[
            jax.ShapeDtypeStruct((bsz, 4 * N, D), jnp.float32),
            jax.ShapeDtypeStruct((bsz, 4 * N, 8), jnp.float32),
        ],
        compiler_params=pltpu.CompilerParams(
            dimension_semantics=("parallel", "parallel")),
    )(gathered, q_pm, x_pm, p_pm, *weights)


def kernel(feature, xyz, bw1, bb1, bw2, bb2, bws, bbs, wq_w, wq_b, wk_w, wk_b,
           wv_w, wv_b, dw1, db1, dg1, dbe1, dw2, db2, gw1, gb1, gg1, gbe1,
           gw2, gb2, aw, ab, m4w1, m4b1, m4w2, m4b2):
    f32 = jnp.float32
    eps = 1e-5

    xyz_pm = jnp.transpose(xyz, (0, 2, 1))
    xyz_p16 = jnp.concatenate(
        [xyz_pm, jnp.zeros((B, N, 13), f32)], axis=2)
    xyz_p8 = xyz_p16[:, :, :8]
    xyz_cm = jnp.concatenate(
        [xyz, jnp.zeros((B, 5, N), f32)], axis=1)

    s1 = dg1 / jnp.sqrt(1.0 + eps)
    sg = gg1 / jnp.sqrt(1.0 + eps)
    dw1t = jnp.concatenate(
        [dw1.T, jnp.zeros((13, D), f32)], axis=0) * s1[None, :]
    db1f = db1 * s1 + dbe1
    gw1t = gw1.T * sg[None, :]
    gb1f = gb1 * sg + gbe1
    awr = jnp.transpose(aw, (2, 0, 1))
    m4w2t = jnp.concatenate(
        [m4w2.T, jnp.zeros((D, 5), f32)], axis=1)
    m4b2p = jnp.concatenate([m4b2, jnp.zeros((5,), f32)])

    knn_w = [bw1.T, bb1[None, :], bw2.T, bb2[None, :], bws.T, bbs[None, :],
             wq_w.T, wq_b[None, :], wk_w.T, wk_b[None, :], wv_w.T, wv_b[None, :],
             dw1t]
    attn_w = [db1f[None, :], dw2.T, db2[None, :], gw1t, gb1f[None, :],
              gw2.T, gb2[None, :], awr, ab[None, :], m4w1.T, m4b1[None, :],
              m4w2t, m4b2p[None, :]]

    H = B // 2
    stage1 = []
    for h in range(2):
        s = slice(h * H, (h + 1) * H)
        stage1.append(_knn_mlp(
            xyz_p8[s], xyz_cm[s], feature[s], xyz_p16[s], knn_w, H))
    gath = []
    for h in range(2):
        idx, _, _, table, _ = stage1[h]
        g = _sc_gather(table.reshape(H * N, TBL),
                       idx.reshape(H * N * K // CHUNK, CHUNK), H)
        gath.append(g.reshape(H, N * K, TBL))
    outs = []
    for h in range(2):
        _, x_pm, q_pm, _, p_pm = stage1[h]
        outs.append(_attn(gath[h], q_pm, x_pm, p_pm, attn_w, H))

    res_pm = jnp.concatenate([o[0] for o in outs], axis=0)
    comp_pm = jnp.concatenate([o[1] for o in outs], axis=0)
    res = jnp.transpose(res_pm, (0, 2, 1))
    completion = jnp.transpose(comp_pm[:, :, :3], (0, 2, 1))
    return (completion, res)
